# Initial kernel scaffold; baseline (speedup 1.0000x reference)
import jax
import jax.numpy as jnp
from jax.experimental import pallas as pl


def kernel(queries, keys, values):
    raise NotImplementedError("write your pallas kernel here")


# speedup vs baseline: 7.3174x; 7.3174x over previous
import numpy as np
import jax
import jax.numpy as jnp
from jax.experimental import pallas as pl
from jax.experimental.pallas import tpu as pltpu

_B, _L, _H, _D = 2, 2048, 12, 64
_BH = _B * _H
_U = 40
_UP = 48
_NT = 4
_TR = _L // _NT
_NEG = -1.0e30


def _build_sample_constants():
    idx = np.asarray(jax.random.randint(jax.random.key(42), (_L, _U), 0, _L))
    counts = np.zeros((_L, _L), np.float32)
    np.add.at(counts, (np.arange(_L)[:, None], idx), 1.0)
    maskbias = np.where(counts > 0, 0.0, _NEG).astype(np.float32)
    return maskbias, counts.astype(jnp.bfloat16)


_MASKBIAS, _ACOUNT = _build_sample_constants()


def _phase_a(q_ref, k_ref, mb_ref, a_ref, m_ref):
    j = pl.program_id(1)
    q = q_ref[0]
    k = k_ref[0]
    kb = k.astype(jnp.bfloat16)
    s = jax.lax.dot_general(q.astype(jnp.bfloat16), kb,
                            (((1,), (1,)), ((), ())),
                            preferred_element_type=jnp.float32)
    mx = jnp.max(s + mb_ref[...], axis=1)
    ksum = jax.lax.dot_general(a_ref[...], kb, (((1,), (0,)), ((), ())),
                               preferred_element_type=jnp.float32)
    ssum = jnp.sum(q * ksum, axis=1)
    m_ref[0, pl.ds(j, 1), :] = (mx - ssum * (1.0 / _L)).reshape(1, _TR)


def _phase_b(m_ref, idx_ref, ms_ref):
    ms_ref[...] = jnp.concatenate(
        [m_ref[i] for i in range(_NT)], axis=1)
    iota = jax.lax.broadcasted_iota(jnp.int32, (_BH, _L), 1).astype(jnp.float32)
    lane = jax.lax.broadcasted_iota(jnp.int32, (_BH, 128), 1)

    def body(t, idxs):
        mval = ms_ref[...]
        mxv = jnp.max(mval, axis=1, keepdims=True)
        cand = jnp.where(mval == mxv, iota, float(_L))
        amin = jnp.min(cand, axis=1, keepdims=True)
        ms_ref[...] = jnp.where(cand == amin, _NEG, mval)
        return jnp.where(lane == t, amin.astype(jnp.int32), idxs)

    idx_ref[...] = jax.lax.fori_loop(
        0, _U, body, jnp.zeros((_BH, 128), jnp.int32))


def _phase_c(idx_ref, q_ref, k_ref, v_ref, o_ref, qr_ref, up_ref):
    j = pl.program_id(0)
    qr_ref[...] = jnp.zeros((_UP, _D), jnp.float32)

    def gat(s, c):
        r = idx_ref[j, s]
        qr_ref[pl.ds(s, 1), :] = q_ref[0, pl.ds(r, 1), :]
        return c

    jax.lax.fori_loop(0, _U, gat, 0)
    k = k_ref[0]
    v = v_ref[0]
    sc = jax.lax.dot_general(qr_ref[...], k, (((1,), (1,)), ((), ())),
                             precision=jax.lax.Precision.HIGHEST) * 0.125
    sc = sc - jnp.max(sc, axis=1, keepdims=True)
    e = jnp.exp(sc)
    att = e / jnp.sum(e, axis=1, keepdims=True)
    up_ref[...] = jax.lax.dot_general(att, v, (((1,), (0,)), ((), ())),
                                      precision=jax.lax.Precision.HIGHEST)
    vm = jnp.sum(v, axis=0, keepdims=True) * (1.0 / _L)
    o_ref[0] = jnp.broadcast_to(vm, (_L, _D))

    def scat(t, c):
        r = idx_ref[j, t]
        o_ref[0, pl.ds(r, 1), :] = up_ref[pl.ds(t, 1), :]
        return c

    jax.lax.fori_loop(0, _U, scat, 0)


def kernel(queries, keys, values):
    q3 = queries.reshape(_BH, _L, _D)
    k3 = keys.reshape(_BH, _L, _D)
    v3 = values.reshape(_BH, _L, _D)
    mb = jnp.asarray(_MASKBIAS)
    ac = jnp.asarray(_ACOUNT)

    m_blk = pl.pallas_call(
        _phase_a,
        grid=(_NT, _BH),
        in_specs=[
            pl.BlockSpec((1, _TR, _D), lambda i, j: (j, i, 0)),
            pl.BlockSpec((1, _L, _D), lambda i, j: (j, 0, 0)),
            pl.BlockSpec((_TR, _L), lambda i, j: (i, 0)),
            pl.BlockSpec((_TR, _L), lambda i, j: (i, 0)),
        ],
        out_specs=pl.BlockSpec((1, _BH, _TR), lambda i, j: (i, 0, 0)),
        out_shape=jax.ShapeDtypeStruct((_NT, _BH, _TR), jnp.float32),
    )(q3, k3, mb, ac)

    idx = pl.pallas_call(
        _phase_b,
        grid=(1,),
        in_specs=[pl.BlockSpec((_NT, _BH, _TR), lambda i: (0, 0, 0))],
        out_specs=pl.BlockSpec((_BH, 128), lambda i: (0, 0)),
        out_shape=jax.ShapeDtypeStruct((_BH, 128), jnp.int32),
        scratch_shapes=[pltpu.VMEM((_BH, _L), jnp.float32)],
    )(m_blk)

    ctx = pl.pallas_call(
        _phase_c,
        grid=(_BH,),
        in_specs=[
            pl.BlockSpec(memory_space=pltpu.SMEM),
            pl.BlockSpec((1, _L, _D), lambda j: (j, 0, 0)),
            pl.BlockSpec((1, _L, _D), lambda j: (j, 0, 0)),
            pl.BlockSpec((1, _L, _D), lambda j: (j, 0, 0)),
        ],
        out_specs=pl.BlockSpec((1, _L, _D), lambda j: (j, 0, 0)),
        out_shape=jax.ShapeDtypeStruct((_BH, _L, _D), jnp.float32),
        scratch_shapes=[
            pltpu.VMEM((_UP, _D), jnp.float32),
            pltpu.VMEM((_UP, _D), jnp.float32),
        ],
    )(idx, q3, k3, v3)

    return ctx.reshape(_B, _H, _L, _D)


# speedup vs baseline: 8.3837x; 1.1457x over previous
import numpy as np
import jax
import jax.numpy as jnp
from jax.experimental import pallas as pl
from jax.experimental.pallas import tpu as pltpu

_B, _L, _H, _D = 2, 2048, 12, 64
_BH = _B * _H
_U = 40
_UP = 48
_NT = 4
_TR = _L // _NT
_NEG = -1.0e30


def _rotl32(x, d):
    return ((x << np.uint32(d)) | (x >> np.uint32(32 - d))).astype(np.uint32)


def _threefry2x32(k1, k2, x0, x1):
    rot = [np.array([13, 15, 26, 6]), np.array([17, 29, 16, 24])]
    ks = [k1, k2, (k1 ^ k2 ^ np.uint32(0x1BD11BDA)).astype(np.uint32)]
    x0 = (x0 + ks[0]).astype(np.uint32)
    x1 = (x1 + ks[1]).astype(np.uint32)
    for i in range(5):
        for r in rot[i % 2]:
            x0 = (x0 + x1).astype(np.uint32)
            x1 = _rotl32(x1, r)
            x1 = (x0 ^ x1).astype(np.uint32)
        x0 = (x0 + ks[(i + 1) % 3]).astype(np.uint32)
        x1 = (x1 + ks[(i + 2) % 3] + np.uint32(i + 1)).astype(np.uint32)
    return x0, x1


def _np_randint_key42(shape, span):
    size = int(np.prod(shape))
    kb1, kb2 = _threefry2x32(np.uint32(0), np.uint32(42),
                             np.zeros(2, np.uint32), np.arange(2, dtype=np.uint32))
    k2a, k2b = kb1[1], kb2[1]
    b1, b2 = _threefry2x32(k2a, k2b, np.zeros(size, np.uint32),
                           np.arange(size, dtype=np.uint32))
    return ((b1 ^ b2) % np.uint32(span)).astype(np.int32).reshape(shape)


def _build_sample_constants():
    idx = _np_randint_key42((_L, _U), _L)
    counts = np.zeros((_L, _L), np.float32)
    np.add.at(counts, (np.arange(_L)[:, None], idx), 1.0)
    maskbias_t = np.where(counts > 0, 0.0, _NEG).astype(np.float32).T.copy()
    return maskbias_t, counts.astype(jnp.bfloat16)


_MASKBIAS, _ACOUNT = _build_sample_constants()


def _phase_a(q_ref, k_ref, mbT_ref, a_ref, m_ref):
    j = pl.program_id(1)
    q = q_ref[0]
    k = k_ref[0]
    kb = k.astype(jnp.bfloat16)
    qb = q.astype(jnp.bfloat16)
    sT = jax.lax.dot_general(kb, qb, (((1,), (1,)), ((), ())),
                             preferred_element_type=jnp.float32)
    mx = jnp.max(sT + mbT_ref[...], axis=0)
    ksumT = jax.lax.dot_general(kb, a_ref[...], (((0,), (1,)), ((), ())),
                                preferred_element_type=jnp.float32)
    ssum = jnp.sum(q.T * ksumT, axis=0)
    m_ref[0, pl.ds(j, 1), :] = (mx - ssum * (1.0 / _L))[None, :]


def _phase_b(m_ref, idx_ref, ms_ref):
    ms_ref[...] = jnp.concatenate(
        [m_ref[i] for i in range(_NT)], axis=1)
    iota = jax.lax.broadcasted_iota(jnp.int32, (_BH, _L), 1).astype(jnp.float32)
    lane = jax.lax.broadcasted_iota(jnp.int32, (_BH, 128), 1)

    def body(t, idxs):
        mval = ms_ref[...]
        mxv = jnp.max(mval, axis=1, keepdims=True)
        cand = jnp.where(mval == mxv, iota, float(_L))
        amin = jnp.min(cand, axis=1, keepdims=True)
        ms_ref[...] = jnp.where(cand == amin, _NEG, mval)
        return jnp.where(lane == t, amin.astype(jnp.int32), idxs)

    idx_ref[...] = jax.lax.fori_loop(
        0, _U, body, jnp.zeros((_BH, 128), jnp.int32))


def _phase_c(idx_ref, q_ref, k_ref, v_ref, o_ref, qr_ref, up_ref):
    j = pl.program_id(0)
    qr_ref[...] = jnp.zeros((_UP, _D), jnp.float32)

    for s2 in range(_U):
        r = idx_ref[j, s2]
        qr_ref[pl.ds(s2, 1), :] = q_ref[0, pl.ds(r, 1), :]
    k = k_ref[0]
    v = v_ref[0]
    sc = jax.lax.dot_general(qr_ref[...], k, (((1,), (1,)), ((), ())),
                             precision=jax.lax.Precision.HIGHEST) * 0.125
    sc = sc - jnp.max(sc, axis=1, keepdims=True)
    e = jnp.exp(sc)
    att = e / jnp.sum(e, axis=1, keepdims=True)
    up_ref[...] = jax.lax.dot_general(att, v, (((1,), (0,)), ((), ())),
                                      precision=jax.lax.Precision.HIGHEST)
    vm = jnp.sum(v, axis=0, keepdims=True) * (1.0 / _L)
    o_ref[0] = jnp.broadcast_to(vm, (_L, _D))

    for t in range(_U):
        r = idx_ref[j, t]
        o_ref[0, pl.ds(r, 1), :] = up_ref[pl.ds(t, 1), :]


def kernel(queries, keys, values):
    q3 = queries.reshape(_BH, _L, _D)
    k3 = keys.reshape(_BH, _L, _D)
    v3 = values.reshape(_BH, _L, _D)
    mb = jnp.asarray(_MASKBIAS)
    ac = jnp.asarray(_ACOUNT)

    m_blk = pl.pallas_call(
        _phase_a,
        grid=(_NT, _BH),
        in_specs=[
            pl.BlockSpec((1, _TR, _D), lambda i, j: (j, i, 0)),
            pl.BlockSpec((1, _L, _D), lambda i, j: (j, 0, 0)),
            pl.BlockSpec((_L, _TR), lambda i, j: (0, i)),
            pl.BlockSpec((_TR, _L), lambda i, j: (i, 0)),
        ],
        out_specs=pl.BlockSpec((1, _BH, _TR), lambda i, j: (i, 0, 0)),
        out_shape=jax.ShapeDtypeStruct((_NT, _BH, _TR), jnp.float32),
    )(q3, k3, mb, ac)

    idx = pl.pallas_call(
        _phase_b,
        grid=(1,),
        in_specs=[pl.BlockSpec((_NT, _BH, _TR), lambda i: (0, 0, 0))],
        out_specs=pl.BlockSpec((_BH, 128), lambda i: (0, 0)),
        out_shape=jax.ShapeDtypeStruct((_BH, 128), jnp.int32),
        scratch_shapes=[pltpu.VMEM((_BH, _L), jnp.float32)],
    )(m_blk)

    ctx = pl.pallas_call(
        _phase_c,
        grid=(_BH,),
        in_specs=[
            pl.BlockSpec(memory_space=pltpu.SMEM),
            pl.BlockSpec((1, _L, _D), lambda j: (j, 0, 0)),
            pl.BlockSpec((1, _L, _D), lambda j: (j, 0, 0)),
            pl.BlockSpec((1, _L, _D), lambda j: (j, 0, 0)),
        ],
        out_specs=pl.BlockSpec((1, _L, _D), lambda j: (j, 0, 0)),
        out_shape=jax.ShapeDtypeStruct((_BH, _L, _D), jnp.float32),
        scratch_shapes=[
            pltpu.VMEM((_UP, _D), jnp.float32),
            pltpu.VMEM((_UP, _D), jnp.float32),
        ],
    )(idx, q3, k3, v3)

    return ctx.reshape(_B, _H, _L, _D)


# speedup vs baseline: 8.7991x; 1.0495x over previous
import numpy as np
import jax
import jax.numpy as jnp
from jax.experimental import pallas as pl
from jax.experimental.pallas import tpu as pltpu

_B, _L, _H, _D = 2, 2048, 12, 64
_BH = _B * _H
_U = 40
_UP = 48
_NT = 4
_TR = _L // _NT
_NEG = -1.0e30


def _rotl32(x, d):
    return ((x << np.uint32(d)) | (x >> np.uint32(32 - d))).astype(np.uint32)


def _threefry2x32(k1, k2, x0, x1):
    rot = [np.array([13, 15, 26, 6]), np.array([17, 29, 16, 24])]
    ks = [k1, k2, (k1 ^ k2 ^ np.uint32(0x1BD11BDA)).astype(np.uint32)]
    x0 = (x0 + ks[0]).astype(np.uint32)
    x1 = (x1 + ks[1]).astype(np.uint32)
    for i in range(5):
        for r in rot[i % 2]:
            x0 = (x0 + x1).astype(np.uint32)
            x1 = _rotl32(x1, r)
            x1 = (x0 ^ x1).astype(np.uint32)
        x0 = (x0 + ks[(i + 1) % 3]).astype(np.uint32)
        x1 = (x1 + ks[(i + 2) % 3] + np.uint32(i + 1)).astype(np.uint32)
    return x0, x1


def _np_randint_key42(shape, span):
    size = int(np.prod(shape))
    kb1, kb2 = _threefry2x32(np.uint32(0), np.uint32(42),
                             np.zeros(2, np.uint32), np.arange(2, dtype=np.uint32))
    k2a, k2b = kb1[1], kb2[1]
    b1, b2 = _threefry2x32(k2a, k2b, np.zeros(size, np.uint32),
                           np.arange(size, dtype=np.uint32))
    return ((b1 ^ b2) % np.uint32(span)).astype(np.int32).reshape(shape)


def _build_sample_constants():
    idx = _np_randint_key42((_L, _U), _L)
    counts = np.zeros((_L, _L), np.float32)
    np.add.at(counts, (np.arange(_L)[:, None], idx), 1.0)
    maskbias_t = np.where(counts > 0, 0.0, _NEG).astype(np.float32).T.copy()
    return maskbias_t, counts.astype(jnp.bfloat16)


_MASKBIAS, _ACOUNT = _build_sample_constants()


def _phase_a(q_ref, k_ref, mbT_ref, a_ref, m_ref):
    j = pl.program_id(1)
    q = q_ref[0]
    k = k_ref[0]
    kb = k.astype(jnp.bfloat16)
    qb = q.astype(jnp.bfloat16)
    sT = jax.lax.dot_general(kb, qb, (((1,), (1,)), ((), ())),
                             preferred_element_type=jnp.float32)
    mx = jnp.max(sT + mbT_ref[...], axis=0)
    ksumT = jax.lax.dot_general(kb, a_ref[...], (((0,), (1,)), ((), ())),
                                preferred_element_type=jnp.float32)
    ssum = jnp.sum(q.T * ksumT, axis=0)
    m_ref[0, pl.ds(j, 1), :] = (mx - ssum * (1.0 / _L))[None, :]


def _phase_b(m_ref, idx_ref, ms_ref):
    ms_ref[...] = jnp.concatenate(
        [m_ref[i] for i in range(_NT)], axis=1)
    iota = jax.lax.broadcasted_iota(jnp.int32, (_BH, _L), 1).astype(jnp.float32)
    lane = jax.lax.broadcasted_iota(jnp.int32, (_BH, 128), 1)

    def body(t, idxs):
        mval = ms_ref[...]
        mxv = jnp.max(mval, axis=1, keepdims=True)
        cand = jnp.where(mval == mxv, iota, float(_L))
        amin = jnp.min(cand, axis=1, keepdims=True)
        ms_ref[...] = jnp.where(cand == amin, _NEG, mval)
        return jnp.where(lane == t, amin.astype(jnp.int32), idxs)

    idx_ref[...] = jax.lax.fori_loop(
        0, _U, body, jnp.zeros((_BH, 128), jnp.int32))


def _phase_c(idx_ref, q_ref, k_ref, v_ref, o_ref, qr_ref, up_ref):
    j = pl.program_id(0)
    qr_ref[...] = jnp.zeros((_UP, _D), jnp.float32)

    for s2 in range(_U):
        r = idx_ref[j, s2]
        qr_ref[pl.ds(s2, 1), :] = q_ref[0, pl.ds(r, 1), :]
    k = k_ref[0]
    v = v_ref[0]
    sc = jax.lax.dot_general(qr_ref[...].astype(jnp.bfloat16),
                             k.astype(jnp.bfloat16), (((1,), (1,)), ((), ())),
                             preferred_element_type=jnp.float32) * 0.125
    sc = sc - jnp.max(sc, axis=1, keepdims=True)
    e = jnp.exp(sc)
    att = e / jnp.sum(e, axis=1, keepdims=True)
    up_ref[...] = jax.lax.dot_general(att.astype(jnp.bfloat16),
                                      v.astype(jnp.bfloat16),
                                      (((1,), (0,)), ((), ())),
                                      preferred_element_type=jnp.float32)
    vm = jnp.sum(v, axis=0, keepdims=True) * (1.0 / _L)
    o_ref[0] = jnp.broadcast_to(vm, (_L, _D))

    for t in range(_U):
        r = idx_ref[j, t]
        o_ref[0, pl.ds(r, 1), :] = up_ref[pl.ds(t, 1), :]


def kernel(queries, keys, values):
    q3 = queries.reshape(_BH, _L, _D)
    k3 = keys.reshape(_BH, _L, _D)
    v3 = values.reshape(_BH, _L, _D)
    mb = jnp.asarray(_MASKBIAS)
    ac = jnp.asarray(_ACOUNT)

    m_blk = pl.pallas_call(
        _phase_a,
        grid=(_NT, _BH),
        in_specs=[
            pl.BlockSpec((1, _TR, _D), lambda i, j: (j, i, 0)),
            pl.BlockSpec((1, _L, _D), lambda i, j: (j, 0, 0)),
            pl.BlockSpec((_L, _TR), lambda i, j: (0, i)),
            pl.BlockSpec((_TR, _L), lambda i, j: (i, 0)),
        ],
        out_specs=pl.BlockSpec((1, _BH, _TR), lambda i, j: (i, 0, 0)),
        out_shape=jax.ShapeDtypeStruct((_NT, _BH, _TR), jnp.float32),
    )(q3, k3, mb, ac)

    idx = pl.pallas_call(
        _phase_b,
        grid=(1,),
        in_specs=[pl.BlockSpec((_NT, _BH, _TR), lambda i: (0, 0, 0))],
        out_specs=pl.BlockSpec((_BH, 128), lambda i: (0, 0)),
        out_shape=jax.ShapeDtypeStruct((_BH, 128), jnp.int32),
        scratch_shapes=[pltpu.VMEM((_BH, _L), jnp.float32)],
    )(m_blk)

    ctx = pl.pallas_call(
        _phase_c,
        grid=(_BH,),
        in_specs=[
            pl.BlockSpec(memory_space=pltpu.SMEM),
            pl.BlockSpec((1, _L, _D), lambda j: (j, 0, 0)),
            pl.BlockSpec((1, _L, _D), lambda j: (j, 0, 0)),
            pl.BlockSpec((1, _L, _D), lambda j: (j, 0, 0)),
        ],
        out_specs=pl.BlockSpec((1, _L, _D), lambda j: (j, 0, 0)),
        out_shape=jax.ShapeDtypeStruct((_BH, _L, _D), jnp.float32),
        scratch_shapes=[
            pltpu.VMEM((_UP, _D), jnp.float32),
            pltpu.VMEM((_UP, _D), jnp.float32),
        ],
    )(idx, q3, k3, v3)

    return ctx.reshape(_B, _H, _L, _D)


# speedup vs baseline: 9.1153x; 1.0359x over previous
import numpy as np
import jax
import jax.numpy as jnp
from jax.experimental import pallas as pl
from jax.experimental.pallas import tpu as pltpu

_B, _L, _H, _D = 2, 2048, 12, 64
_BH = _B * _H
_U = 40
_UP = 48
_NT = 2
_TR = _L // _NT
_NEG = -1.0e30


def _rotl32(x, d):
    return ((x << np.uint32(d)) | (x >> np.uint32(32 - d))).astype(np.uint32)


def _threefry2x32(k1, k2, x0, x1):
    rot = [np.array([13, 15, 26, 6]), np.array([17, 29, 16, 24])]
    ks = [k1, k2, (k1 ^ k2 ^ np.uint32(0x1BD11BDA)).astype(np.uint32)]
    x0 = (x0 + ks[0]).astype(np.uint32)
    x1 = (x1 + ks[1]).astype(np.uint32)
    for i in range(5):
        for r in rot[i % 2]:
            x0 = (x0 + x1).astype(np.uint32)
            x1 = _rotl32(x1, r)
            x1 = (x0 ^ x1).astype(np.uint32)
        x0 = (x0 + ks[(i + 1) % 3]).astype(np.uint32)
        x1 = (x1 + ks[(i + 2) % 3] + np.uint32(i + 1)).astype(np.uint32)
    return x0, x1


def _np_randint_key42(shape, span):
    size = int(np.prod(shape))
    kb1, kb2 = _threefry2x32(np.uint32(0), np.uint32(42),
                             np.zeros(2, np.uint32), np.arange(2, dtype=np.uint32))
    k2a, k2b = kb1[1], kb2[1]
    b1, b2 = _threefry2x32(k2a, k2b, np.zeros(size, np.uint32),
                           np.arange(size, dtype=np.uint32))
    return ((b1 ^ b2) % np.uint32(span)).astype(np.int32).reshape(shape)


def _build_sample_constants():
    idx = _np_randint_key42((_L, _U), _L)
    counts = np.zeros((_L, _L), np.float32)
    np.add.at(counts, (np.arange(_L)[:, None], idx), 1.0)
    maskbias_t = np.where(counts > 0, 0.0, _NEG).astype(np.float32).T.copy()
    return maskbias_t, counts.astype(jnp.bfloat16)


_MASKBIAS, _ACOUNT = _build_sample_constants()


def _phase_a(q_ref, k_ref, mbT_ref, a_ref, m_ref):
    j = pl.program_id(1)
    q = q_ref[0]
    k = k_ref[0]
    kb = k.astype(jnp.bfloat16)
    qb = q.astype(jnp.bfloat16)
    sT = jax.lax.dot_general(kb, qb, (((1,), (1,)), ((), ())),
                             preferred_element_type=jnp.float32)
    mx = jnp.max(sT + mbT_ref[...], axis=0)
    ksumT = jax.lax.dot_general(kb, a_ref[...], (((0,), (1,)), ((), ())),
                                preferred_element_type=jnp.float32)
    ssum = jnp.sum(q.T * ksumT, axis=0)
    m_ref[0, pl.ds(j, 1), :] = (mx - ssum * (1.0 / _L))[None, :]


def _phase_b(m_ref, idx_ref, ms_ref):
    ms_ref[...] = jnp.concatenate(
        [m_ref[i] for i in range(_NT)], axis=1)
    iota = jax.lax.broadcasted_iota(jnp.int32, (_BH, _L), 1).astype(jnp.float32)
    lane = jax.lax.broadcasted_iota(jnp.int32, (_BH, 128), 1)

    def body(t, idxs):
        mval = ms_ref[...]
        mxv = jnp.max(mval, axis=1, keepdims=True)
        cand = jnp.where(mval == mxv, iota, float(_L))
        amin = jnp.min(cand, axis=1, keepdims=True)
        ms_ref[...] = jnp.where(cand == amin, _NEG, mval)
        return jnp.where(lane == t, amin.astype(jnp.int32), idxs)

    idx_ref[...] = jax.lax.fori_loop(
        0, _U, body, jnp.zeros((_BH, 128), jnp.int32))


def _phase_c(idx_ref, q_ref, k_ref, v_ref, o_ref, qr_ref, up_ref):
    j = pl.program_id(0)
    qr_ref[...] = jnp.zeros((_UP, _D), jnp.float32)

    for s2 in range(_U):
        r = idx_ref[j, s2]
        qr_ref[pl.ds(s2, 1), :] = q_ref[0, pl.ds(r, 1), :]
    k = k_ref[0]
    v = v_ref[0]
    sc = jax.lax.dot_general(qr_ref[...].astype(jnp.bfloat16),
                             k.astype(jnp.bfloat16), (((1,), (1,)), ((), ())),
                             preferred_element_type=jnp.float32) * 0.125
    sc = sc - jnp.max(sc, axis=1, keepdims=True)
    e = jnp.exp(sc)
    att = e / jnp.sum(e, axis=1, keepdims=True)
    up_ref[...] = jax.lax.dot_general(att.astype(jnp.bfloat16),
                                      v.astype(jnp.bfloat16),
                                      (((1,), (0,)), ((), ())),
                                      preferred_element_type=jnp.float32)
    vm = jnp.sum(v, axis=0, keepdims=True) * (1.0 / _L)
    o_ref[0] = jnp.broadcast_to(vm, (_L, _D))

    for t in range(_U):
        r = idx_ref[j, t]
        o_ref[0, pl.ds(r, 1), :] = up_ref[pl.ds(t, 1), :]


def kernel(queries, keys, values):
    q3 = queries.reshape(_BH, _L, _D)
    k3 = keys.reshape(_BH, _L, _D)
    v3 = values.reshape(_BH, _L, _D)
    mb = jnp.asarray(_MASKBIAS)
    ac = jnp.asarray(_ACOUNT)

    m_blk = pl.pallas_call(
        _phase_a,
        grid=(_NT, _BH),
        in_specs=[
            pl.BlockSpec((1, _TR, _D), lambda i, j: (j, i, 0)),
            pl.BlockSpec((1, _L, _D), lambda i, j: (j, 0, 0)),
            pl.BlockSpec((_L, _TR), lambda i, j: (0, i)),
            pl.BlockSpec((_TR, _L), lambda i, j: (i, 0)),
        ],
        out_specs=pl.BlockSpec((1, _BH, _TR), lambda i, j: (i, 0, 0)),
        out_shape=jax.ShapeDtypeStruct((_NT, _BH, _TR), jnp.float32),
    )(q3, k3, mb, ac)

    idx = pl.pallas_call(
        _phase_b,
        grid=(1,),
        in_specs=[pl.BlockSpec((_NT, _BH, _TR), lambda i: (0, 0, 0))],
        out_specs=pl.BlockSpec((_BH, 128), lambda i: (0, 0)),
        out_shape=jax.ShapeDtypeStruct((_BH, 128), jnp.int32),
        scratch_shapes=[pltpu.VMEM((_BH, _L), jnp.float32)],
    )(m_blk)

    ctx = pl.pallas_call(
        _phase_c,
        grid=(_BH,),
        in_specs=[
            pl.BlockSpec(memory_space=pltpu.SMEM),
            pl.BlockSpec((1, _L, _D), lambda j: (j, 0, 0)),
            pl.BlockSpec((1, _L, _D), lambda j: (j, 0, 0)),
            pl.BlockSpec((1, _L, _D), lambda j: (j, 0, 0)),
        ],
        out_specs=pl.BlockSpec((1, _L, _D), lambda j: (j, 0, 0)),
        out_shape=jax.ShapeDtypeStruct((_BH, _L, _D), jnp.float32),
        scratch_shapes=[
            pltpu.VMEM((_UP, _D), jnp.float32),
            pltpu.VMEM((_UP, _D), jnp.float32),
        ],
    )(idx, q3, k3, v3)

    return ctx.reshape(_B, _H, _L, _D)


# speedup vs baseline: 9.4464x; 1.0363x over previous
import numpy as np
import jax
import jax.numpy as jnp
from jax.experimental import pallas as pl
from jax.experimental.pallas import tpu as pltpu

_B, _L, _H, _D = 2, 2048, 12, 64
_BH = _B * _H
_U = 40
_UP = 48
_NT = 2
_TR = _L // _NT
_NEG = -1.0e30


def _rotl32(x, d):
    return ((x << np.uint32(d)) | (x >> np.uint32(32 - d))).astype(np.uint32)


def _threefry2x32(k1, k2, x0, x1):
    rot = [np.array([13, 15, 26, 6]), np.array([17, 29, 16, 24])]
    ks = [k1, k2, (k1 ^ k2 ^ np.uint32(0x1BD11BDA)).astype(np.uint32)]
    x0 = (x0 + ks[0]).astype(np.uint32)
    x1 = (x1 + ks[1]).astype(np.uint32)
    for i in range(5):
        for r in rot[i % 2]:
            x0 = (x0 + x1).astype(np.uint32)
            x1 = _rotl32(x1, r)
            x1 = (x0 ^ x1).astype(np.uint32)
        x0 = (x0 + ks[(i + 1) % 3]).astype(np.uint32)
        x1 = (x1 + ks[(i + 2) % 3] + np.uint32(i + 1)).astype(np.uint32)
    return x0, x1


def _np_randint_key42(shape, span):
    size = int(np.prod(shape))
    kb1, kb2 = _threefry2x32(np.uint32(0), np.uint32(42),
                             np.zeros(2, np.uint32), np.arange(2, dtype=np.uint32))
    k2a, k2b = kb1[1], kb2[1]
    b1, b2 = _threefry2x32(k2a, k2b, np.zeros(size, np.uint32),
                           np.arange(size, dtype=np.uint32))
    return ((b1 ^ b2) % np.uint32(span)).astype(np.int32).reshape(shape)


def _build_sample_constants():
    idx = _np_randint_key42((_L, _U), _L)
    counts = np.zeros((_L, _L), np.float32)
    np.add.at(counts, (np.arange(_L)[:, None], idx), 1.0)
    return counts.T.copy().astype(jnp.bfloat16)


_ACOUNT_T = _build_sample_constants()


def _phase_a(q_ref, k_ref, at_ref, m_ref):
    j = pl.program_id(1)
    q = q_ref[0]
    k = k_ref[0]
    kb = k.astype(jnp.bfloat16)
    qb = q.astype(jnp.bfloat16)
    at = at_ref[...]
    sT = jax.lax.dot_general(kb, qb, (((1,), (1,)), ((), ())),
                             preferred_element_type=jnp.float32)
    mx = jnp.max(jnp.where(at > 0, sT, _NEG), axis=0)
    ksumT = jax.lax.dot_general(kb, at, (((0,), (0,)), ((), ())),
                                preferred_element_type=jnp.float32)
    ssum = jnp.sum(q.T * ksumT, axis=0)
    m_ref[0, pl.ds(j, 1), :] = (mx - ssum * (1.0 / _L))[None, :]


def _phase_b(m_ref, idx_ref, ms_ref):
    ms_ref[...] = jnp.concatenate(
        [m_ref[i] for i in range(_NT)], axis=1)
    iota = jax.lax.broadcasted_iota(jnp.int32, (_BH, _L), 1).astype(jnp.float32)
    lane = jax.lax.broadcasted_iota(jnp.int32, (_BH, 128), 1)

    def body(t, idxs):
        mval = ms_ref[...]
        mxv = jnp.max(mval, axis=1, keepdims=True)
        cand = jnp.where(mval == mxv, iota, float(_L))
        amin = jnp.min(cand, axis=1, keepdims=True)
        ms_ref[...] = jnp.where(cand == amin, _NEG, mval)
        return jnp.where(lane == t, amin.astype(jnp.int32), idxs)

    idx_ref[...] = jax.lax.fori_loop(
        0, _U, body, jnp.zeros((_BH, 128), jnp.int32))


def _phase_c(idx_ref, q_ref, k_ref, v_ref, o_ref, qr_ref, up_ref):
    j = pl.program_id(0)
    qr_ref[...] = jnp.zeros((_UP, _D), jnp.float32)

    for s2 in range(_U):
        r = idx_ref[j, s2]
        qr_ref[pl.ds(s2, 1), :] = q_ref[0, pl.ds(r, 1), :]
    k = k_ref[0]
    v = v_ref[0]
    sc = jax.lax.dot_general(qr_ref[...].astype(jnp.bfloat16),
                             k.astype(jnp.bfloat16), (((1,), (1,)), ((), ())),
                             preferred_element_type=jnp.float32) * 0.125
    sc = sc - jnp.max(sc, axis=1, keepdims=True)
    e = jnp.exp(sc)
    att = e / jnp.sum(e, axis=1, keepdims=True)
    up_ref[...] = jax.lax.dot_general(att.astype(jnp.bfloat16),
                                      v.astype(jnp.bfloat16),
                                      (((1,), (0,)), ((), ())),
                                      preferred_element_type=jnp.float32)
    vm = jnp.sum(v, axis=0, keepdims=True) * (1.0 / _L)
    o_ref[0] = jnp.broadcast_to(vm, (_L, _D))

    for t in range(_U):
        r = idx_ref[j, t]
        o_ref[0, pl.ds(r, 1), :] = up_ref[pl.ds(t, 1), :]


def kernel(queries, keys, values):
    q3 = queries.reshape(_BH, _L, _D)
    k3 = keys.reshape(_BH, _L, _D)
    v3 = values.reshape(_BH, _L, _D)
    at = jnp.asarray(_ACOUNT_T)

    m_blk = pl.pallas_call(
        _phase_a,
        grid=(_NT, _BH),
        in_specs=[
            pl.BlockSpec((1, _TR, _D), lambda i, j: (j, i, 0)),
            pl.BlockSpec((1, _L, _D), lambda i, j: (j, 0, 0)),
            pl.BlockSpec((_L, _TR), lambda i, j: (0, i)),
        ],
        out_specs=pl.BlockSpec((1, _BH, _TR), lambda i, j: (i, 0, 0)),
        out_shape=jax.ShapeDtypeStruct((_NT, _BH, _TR), jnp.float32),
    )(q3, k3, at)

    idx = pl.pallas_call(
        _phase_b,
        grid=(1,),
        in_specs=[pl.BlockSpec((_NT, _BH, _TR), lambda i: (0, 0, 0))],
        out_specs=pl.BlockSpec((_BH, 128), lambda i: (0, 0)),
        out_shape=jax.ShapeDtypeStruct((_BH, 128), jnp.int32),
        scratch_shapes=[pltpu.VMEM((_BH, _L), jnp.float32)],
    )(m_blk)

    ctx = pl.pallas_call(
        _phase_c,
        grid=(_BH,),
        in_specs=[
            pl.BlockSpec(memory_space=pltpu.SMEM),
            pl.BlockSpec((1, _L, _D), lambda j: (j, 0, 0)),
            pl.BlockSpec((1, _L, _D), lambda j: (j, 0, 0)),
            pl.BlockSpec((1, _L, _D), lambda j: (j, 0, 0)),
        ],
        out_specs=pl.BlockSpec((1, _L, _D), lambda j: (j, 0, 0)),
        out_shape=jax.ShapeDtypeStruct((_BH, _L, _D), jnp.float32),
        scratch_shapes=[
            pltpu.VMEM((_UP, _D), jnp.float32),
            pltpu.VMEM((_UP, _D), jnp.float32),
        ],
    )(idx, q3, k3, v3)

    return ctx.reshape(_B, _H, _L, _D)


# speedup vs baseline: 9.7352x; 1.0306x over previous
import functools
import numpy as np
import jax
import jax.numpy as jnp
from jax import lax
from jax.experimental import pallas as pl
from jax.experimental.pallas import tpu as pltpu
from jax.experimental.pallas import tpu_sc as plsc

_B, _L, _H, _D = 2, 2048, 12, 64
_BH = _B * _H
_U = 40
_UP = 48
_NT = 2
_TR = _L // _NT
_NEG = -1.0e30


def _rotl32(x, d):
    return ((x << np.uint32(d)) | (x >> np.uint32(32 - d))).astype(np.uint32)


def _threefry2x32(k1, k2, x0, x1):
    rot = [np.array([13, 15, 26, 6]), np.array([17, 29, 16, 24])]
    ks = [k1, k2, (k1 ^ k2 ^ np.uint32(0x1BD11BDA)).astype(np.uint32)]
    x0 = (x0 + ks[0]).astype(np.uint32)
    x1 = (x1 + ks[1]).astype(np.uint32)
    for i in range(5):
        for r in rot[i % 2]:
            x0 = (x0 + x1).astype(np.uint32)
            x1 = _rotl32(x1, r)
            x1 = (x0 ^ x1).astype(np.uint32)
        x0 = (x0 + ks[(i + 1) % 3]).astype(np.uint32)
        x1 = (x1 + ks[(i + 2) % 3] + np.uint32(i + 1)).astype(np.uint32)
    return x0, x1


def _np_randint_key42(shape, span):
    size = int(np.prod(shape))
    kb1, kb2 = _threefry2x32(np.uint32(0), np.uint32(42),
                             np.zeros(2, np.uint32), np.arange(2, dtype=np.uint32))
    k2a, k2b = kb1[1], kb2[1]
    b1, b2 = _threefry2x32(k2a, k2b, np.zeros(size, np.uint32),
                           np.arange(size, dtype=np.uint32))
    return ((b1 ^ b2) % np.uint32(span)).astype(np.int32).reshape(shape)


def _build_sample_constants():
    idx = _np_randint_key42((_L, _U), _L)
    counts = np.zeros((_L, _L), np.float32)
    np.add.at(counts, (np.arange(_L)[:, None], idx), 1.0)
    return counts.T.copy().astype(jnp.bfloat16)


_ACOUNT_T = _build_sample_constants()


def _phase_a(q_ref, k_ref, at_ref, m_ref):
    j = pl.program_id(1)
    q = q_ref[0]
    k = k_ref[0]
    kb = k.astype(jnp.bfloat16)
    qb = q.astype(jnp.bfloat16)
    at = at_ref[...]
    sT = jax.lax.dot_general(kb, qb, (((1,), (1,)), ((), ())),
                             preferred_element_type=jnp.float32)
    mx = jnp.max(jnp.where(at > 0, sT, _NEG), axis=0)
    ksumT = jax.lax.dot_general(kb, at, (((0,), (0,)), ((), ())),
                                preferred_element_type=jnp.float32)
    ssum = jnp.sum(q.T * ksumT, axis=0)
    m_ref[0, pl.ds(j, 1), :] = (mx - ssum * (1.0 / _L))[None, :]


def _bmax_f32(v, rbuf):
    for sh in (8, 4, 2, 1):
        rbuf[pl.ds(0, 16)] = v
        rbuf[pl.ds(16, 16)] = v
        v = jnp.maximum(v, rbuf[pl.ds(sh, 16)])
    return v


def _bmin_i32(v, ibuf):
    for sh in (8, 4, 2, 1):
        ibuf[pl.ds(0, 16)] = v
        ibuf[pl.ds(16, 16)] = v
        v = jnp.minimum(v, ibuf[pl.ds(sh, 16)])
    return v


def _sc_topk(m_hbm, idx_hbm, mrow, idxrow, rbuf, ibuf):
    wid = lax.axis_index("s") * 2 + lax.axis_index("c")
    lanes = lax.iota(jnp.int32, 16)
    negv = jnp.full((16,), _NEG, jnp.float32)

    @pl.when(wid < _BH)
    def _():
        for i in range(_NT):
            pltpu.sync_copy(m_hbm.at[i, wid], mrow.at[pl.ds(i * _TR, _TR)])

        def t_body(t, carry):
            gmp, lip = carry

            def c_body(c, cc):
                bestv, besti = cc
                v = mrow[pl.ds(c * 16, 16)]
                idxv = lanes + c * 16
                elig = (v < gmp) | ((v == gmp) & (idxv > lip))
                veff = jnp.where(elig, v, negv)
                better = veff > bestv
                return (jnp.where(better, veff, bestv),
                        jnp.where(better, idxv, besti))

            bestv, besti = lax.fori_loop(
                0, _L // 16, c_body, (negv, jnp.zeros((16,), jnp.int32)))
            gm = _bmax_f32(bestv, rbuf)
            cand = jnp.where(bestv == gm, besti,
                             jnp.full((16,), 4096, jnp.int32))
            li = _bmin_i32(cand, ibuf)
            base = (t // 16) * 16
            off = t % 16
            old = idxrow[pl.ds(base, 16)]
            idxrow[pl.ds(base, 16)] = jnp.where(lanes == off, li, old)
            return gm, li

        lax.fori_loop(0, _U, t_body,
                      (jnp.full((16,), 3.0e38, jnp.float32),
                       jnp.full((16,), -1, jnp.int32)))

        pltpu.sync_copy(idxrow, idx_hbm.at[wid])


def _sc_topk_call(m_blk):
    mesh = plsc.VectorSubcoreMesh(core_axis_name="c", subcore_axis_name="s")
    f = functools.partial(
        pl.kernel, mesh=mesh,
        out_type=jax.ShapeDtypeStruct((_BH, 128), jnp.int32),
        scratch_types=[
            pltpu.VMEM((_L,), jnp.float32),
            pltpu.VMEM((128,), jnp.int32),
            pltpu.VMEM((32,), jnp.float32),
            pltpu.VMEM((32,), jnp.int32),
        ],
    )(_sc_topk)
    return f(m_blk)


def _phase_c(idx_ref, q_ref, k_ref, v_ref, o_ref, qr_ref, up_ref):
    j = pl.program_id(0)
    for s2 in range(_U):
        r = idx_ref[j, s2]
        qr_ref[pl.ds(s2, 1), :] = q_ref[0, pl.ds(r, 1), :]
    k = k_ref[0]
    v = v_ref[0]
    sc = jax.lax.dot_general(qr_ref[...].astype(jnp.bfloat16),
                             k.astype(jnp.bfloat16), (((1,), (1,)), ((), ())),
                             preferred_element_type=jnp.float32) * 0.125
    sc = sc - jnp.max(sc, axis=1, keepdims=True)
    e = jnp.exp(sc)
    att = e / jnp.sum(e, axis=1, keepdims=True)
    up_ref[...] = jax.lax.dot_general(att.astype(jnp.bfloat16),
                                      v.astype(jnp.bfloat16),
                                      (((1,), (0,)), ((), ())),
                                      preferred_element_type=jnp.float32)
    vm = jnp.sum(v, axis=0, keepdims=True) * (1.0 / _L)
    o_ref[0] = jnp.broadcast_to(vm, (_L, _D))

    for t in range(_U):
        r = idx_ref[j, t]
        o_ref[0, pl.ds(r, 1), :] = up_ref[pl.ds(t, 1), :]


def kernel(queries, keys, values):
    q3 = queries.reshape(_BH, _L, _D)
    k3 = keys.reshape(_BH, _L, _D)
    v3 = values.reshape(_BH, _L, _D)
    at = jnp.asarray(_ACOUNT_T)

    m_blk = pl.pallas_call(
        _phase_a,
        grid=(_NT, _BH),
        in_specs=[
            pl.BlockSpec((1, _TR, _D), lambda i, j: (j, i, 0)),
            pl.BlockSpec((1, _L, _D), lambda i, j: (j, 0, 0)),
            pl.BlockSpec((_L, _TR), lambda i, j: (0, i)),
        ],
        out_specs=pl.BlockSpec((1, _BH, _TR), lambda i, j: (i, 0, 0)),
        out_shape=jax.ShapeDtypeStruct((_NT, _BH, _TR), jnp.float32),
    )(q3, k3, at)

    idx = _sc_topk_call(m_blk)

    ctx = pl.pallas_call(
        _phase_c,
        grid=(_BH,),
        in_specs=[
            pl.BlockSpec(memory_space=pltpu.SMEM),
            pl.BlockSpec((1, _L, _D), lambda j: (j, 0, 0)),
            pl.BlockSpec((1, _L, _D), lambda j: (j, 0, 0)),
            pl.BlockSpec((1, _L, _D), lambda j: (j, 0, 0)),
        ],
        out_specs=pl.BlockSpec((1, _L, _D), lambda j: (j, 0, 0)),
        out_shape=jax.ShapeDtypeStruct((_BH, _L, _D), jnp.float32),
        scratch_shapes=[
            pltpu.VMEM((_U, _D), jnp.float32),
            pltpu.VMEM((_U, _D), jnp.float32),
        ],
    )(idx, q3, k3, v3)

    return ctx.reshape(_B, _H, _L, _D)


# speedup vs baseline: 9.8620x; 1.0130x over previous
import functools
import numpy as np
import jax
import jax.numpy as jnp
from jax import lax
from jax.experimental import pallas as pl
from jax.experimental.pallas import tpu as pltpu
from jax.experimental.pallas import tpu_sc as plsc

_B, _L, _H, _D = 2, 2048, 12, 64
_BH = _B * _H
_U = 40
_UP = 48
_NT = 1
_TR = _L // _NT
_NEG = -1.0e30


def _rotl32(x, d):
    return ((x << np.uint32(d)) | (x >> np.uint32(32 - d))).astype(np.uint32)


def _threefry2x32(k1, k2, x0, x1):
    rot = [np.array([13, 15, 26, 6]), np.array([17, 29, 16, 24])]
    ks = [k1, k2, (k1 ^ k2 ^ np.uint32(0x1BD11BDA)).astype(np.uint32)]
    x0 = (x0 + ks[0]).astype(np.uint32)
    x1 = (x1 + ks[1]).astype(np.uint32)
    for i in range(5):
        for r in rot[i % 2]:
            x0 = (x0 + x1).astype(np.uint32)
            x1 = _rotl32(x1, r)
            x1 = (x0 ^ x1).astype(np.uint32)
        x0 = (x0 + ks[(i + 1) % 3]).astype(np.uint32)
        x1 = (x1 + ks[(i + 2) % 3] + np.uint32(i + 1)).astype(np.uint32)
    return x0, x1


def _np_randint_key42(shape, span):
    size = int(np.prod(shape))
    kb1, kb2 = _threefry2x32(np.uint32(0), np.uint32(42),
                             np.zeros(2, np.uint32), np.arange(2, dtype=np.uint32))
    k2a, k2b = kb1[1], kb2[1]
    b1, b2 = _threefry2x32(k2a, k2b, np.zeros(size, np.uint32),
                           np.arange(size, dtype=np.uint32))
    return ((b1 ^ b2) % np.uint32(span)).astype(np.int32).reshape(shape)


def _build_sample_constants():
    idx = _np_randint_key42((_L, _U), _L)
    counts = np.zeros((_L, _L), np.float32)
    np.add.at(counts, (np.arange(_L)[:, None], idx), 1.0)
    return counts.T.copy().astype(jnp.bfloat16)


_ACOUNT_T = _build_sample_constants()


def _phase_a(q_ref, k_ref, at_ref, m_ref):
    j = pl.program_id(1)
    q = q_ref[0]
    k = k_ref[0]
    kb = k.astype(jnp.bfloat16)
    qb = q.astype(jnp.bfloat16)
    at = at_ref[...]
    sT = jax.lax.dot_general(kb, qb, (((1,), (1,)), ((), ())),
                             preferred_element_type=jnp.float32)
    mx = jnp.max(jnp.where(at > 0, sT, _NEG), axis=0)
    ksumT = jax.lax.dot_general(kb, at, (((0,), (0,)), ((), ())),
                                preferred_element_type=jnp.float32)
    ssum = jnp.sum(q.T * ksumT, axis=0)
    m_ref[0, pl.ds(j, 1), :] = (mx - ssum * (1.0 / _L))[None, :]


def _bmax_f32(v, rbuf):
    for sh in (8, 4, 2, 1):
        rbuf[pl.ds(0, 16)] = v
        rbuf[pl.ds(16, 16)] = v
        v = jnp.maximum(v, rbuf[pl.ds(sh, 16)])
    return v


def _bmin_i32(v, ibuf):
    for sh in (8, 4, 2, 1):
        ibuf[pl.ds(0, 16)] = v
        ibuf[pl.ds(16, 16)] = v
        v = jnp.minimum(v, ibuf[pl.ds(sh, 16)])
    return v


def _sc_topk(m_hbm, idx_hbm, mrow, idxrow, rbuf, ibuf):
    wid = lax.axis_index("s") * 2 + lax.axis_index("c")
    lanes = lax.iota(jnp.int32, 16)
    negv = jnp.full((16,), _NEG, jnp.float32)

    @pl.when(wid < _BH)
    def _():
        for i in range(_NT):
            pltpu.sync_copy(m_hbm.at[i, wid], mrow.at[pl.ds(i * _TR, _TR)])

        def t_body(t, carry):
            gmp, lip = carry

            def c_body(c, cc):
                bestv, besti = cc
                v = mrow[pl.ds(c * 16, 16)]
                idxv = lanes + c * 16
                elig = (v < gmp) | ((v == gmp) & (idxv > lip))
                veff = jnp.where(elig, v, negv)
                better = veff > bestv
                return (jnp.where(better, veff, bestv),
                        jnp.where(better, idxv, besti))

            bestv, besti = lax.fori_loop(
                0, _L // 16, c_body, (negv, jnp.zeros((16,), jnp.int32)))
            gm = _bmax_f32(bestv, rbuf)
            cand = jnp.where(bestv == gm, besti,
                             jnp.full((16,), 4096, jnp.int32))
            li = _bmin_i32(cand, ibuf)
            base = (t // 16) * 16
            off = t % 16
            old = idxrow[pl.ds(base, 16)]
            idxrow[pl.ds(base, 16)] = jnp.where(lanes == off, li, old)
            return gm, li

        lax.fori_loop(0, _U, t_body,
                      (jnp.full((16,), 3.0e38, jnp.float32),
                       jnp.full((16,), -1, jnp.int32)))

        pltpu.sync_copy(idxrow, idx_hbm.at[wid])


def _sc_topk_call(m_blk):
    mesh = plsc.VectorSubcoreMesh(core_axis_name="c", subcore_axis_name="s")
    f = functools.partial(
        pl.kernel, mesh=mesh,
        out_type=jax.ShapeDtypeStruct((_BH, 128), jnp.int32),
        scratch_types=[
            pltpu.VMEM((_L,), jnp.float32),
            pltpu.VMEM((128,), jnp.int32),
            pltpu.VMEM((32,), jnp.float32),
            pltpu.VMEM((32,), jnp.int32),
        ],
    )(_sc_topk)
    return f(m_blk)


def _phase_c(idx_ref, q_ref, k_ref, v_ref, o_ref, qr_ref, up_ref):
    j = pl.program_id(0)
    for s2 in range(_U):
        r = idx_ref[j, s2]
        qr_ref[pl.ds(s2, 1), :] = q_ref[0, pl.ds(r, 1), :]
    k = k_ref[0]
    v = v_ref[0]
    sc = jax.lax.dot_general(qr_ref[...].astype(jnp.bfloat16),
                             k.astype(jnp.bfloat16), (((1,), (1,)), ((), ())),
                             preferred_element_type=jnp.float32) * 0.125
    sc = sc - jnp.max(sc, axis=1, keepdims=True)
    e = jnp.exp(sc)
    att = e / jnp.sum(e, axis=1, keepdims=True)
    up_ref[...] = jax.lax.dot_general(att.astype(jnp.bfloat16),
                                      v.astype(jnp.bfloat16),
                                      (((1,), (0,)), ((), ())),
                                      preferred_element_type=jnp.float32)
    vm = jnp.sum(v, axis=0, keepdims=True) * (1.0 / _L)
    o_ref[0] = jnp.broadcast_to(vm, (_L, _D))

    for t in range(_U):
        r = idx_ref[j, t]
        o_ref[0, pl.ds(r, 1), :] = up_ref[pl.ds(t, 1), :]


def kernel(queries, keys, values):
    q3 = queries.reshape(_BH, _L, _D)
    k3 = keys.reshape(_BH, _L, _D)
    v3 = values.reshape(_BH, _L, _D)
    at = jnp.asarray(_ACOUNT_T)

    m_blk = pl.pallas_call(
        _phase_a,
        grid=(_NT, _BH),
        in_specs=[
            pl.BlockSpec((1, _TR, _D), lambda i, j: (j, i, 0)),
            pl.BlockSpec((1, _L, _D), lambda i, j: (j, 0, 0)),
            pl.BlockSpec((_L, _TR), lambda i, j: (0, i)),
        ],
        out_specs=pl.BlockSpec((1, _BH, _TR), lambda i, j: (i, 0, 0)),
        out_shape=jax.ShapeDtypeStruct((_NT, _BH, _TR), jnp.float32),
    )(q3, k3, at)

    idx = _sc_topk_call(m_blk)

    ctx = pl.pallas_call(
        _phase_c,
        grid=(_BH,),
        in_specs=[
            pl.BlockSpec(memory_space=pltpu.SMEM),
            pl.BlockSpec((1, _L, _D), lambda j: (j, 0, 0)),
            pl.BlockSpec((1, _L, _D), lambda j: (j, 0, 0)),
            pl.BlockSpec((1, _L, _D), lambda j: (j, 0, 0)),
        ],
        out_specs=pl.BlockSpec((1, _L, _D), lambda j: (j, 0, 0)),
        out_shape=jax.ShapeDtypeStruct((_BH, _L, _D), jnp.float32),
        scratch_shapes=[
            pltpu.VMEM((_U, _D), jnp.float32),
            pltpu.VMEM((_U, _D), jnp.float32),
        ],
    )(idx, q3, k3, v3)

    return ctx.reshape(_B, _H, _L, _D)


# speedup vs baseline: 10.3639x; 1.0509x over previous
import functools
import numpy as np
import jax
import jax.numpy as jnp
from jax import lax
from jax.experimental import pallas as pl
from jax.experimental.pallas import tpu as pltpu
from jax.experimental.pallas import tpu_sc as plsc

_B, _L, _H, _D = 2, 2048, 12, 64
_BH = _B * _H
_U = 40
_UP = 48
_NT = 1
_TR = _L // _NT
_NEG = -1.0e30


def _rotl32(x, d):
    return ((x << np.uint32(d)) | (x >> np.uint32(32 - d))).astype(np.uint32)


def _threefry2x32(k1, k2, x0, x1):
    rot = [np.array([13, 15, 26, 6]), np.array([17, 29, 16, 24])]
    ks = [k1, k2, (k1 ^ k2 ^ np.uint32(0x1BD11BDA)).astype(np.uint32)]
    x0 = (x0 + ks[0]).astype(np.uint32)
    x1 = (x1 + ks[1]).astype(np.uint32)
    for i in range(5):
        for r in rot[i % 2]:
            x0 = (x0 + x1).astype(np.uint32)
            x1 = _rotl32(x1, r)
            x1 = (x0 ^ x1).astype(np.uint32)
        x0 = (x0 + ks[(i + 1) % 3]).astype(np.uint32)
        x1 = (x1 + ks[(i + 2) % 3] + np.uint32(i + 1)).astype(np.uint32)
    return x0, x1


def _np_randint_key42(shape, span):
    size = int(np.prod(shape))
    kb1, kb2 = _threefry2x32(np.uint32(0), np.uint32(42),
                             np.zeros(2, np.uint32), np.arange(2, dtype=np.uint32))
    k2a, k2b = kb1[1], kb2[1]
    b1, b2 = _threefry2x32(k2a, k2b, np.zeros(size, np.uint32),
                           np.arange(size, dtype=np.uint32))
    return ((b1 ^ b2) % np.uint32(span)).astype(np.int32).reshape(shape)


def _build_sample_constants():
    idx = _np_randint_key42((_L, _U), _L)
    counts = np.zeros((_L, _L), np.float32)
    np.add.at(counts, (np.arange(_L)[:, None], idx), 1.0)
    counts_t = counts.T.copy()
    maskbias_t = np.where(counts_t > 0, 0.0, _NEG).astype(np.float32)
    return counts_t.astype(jnp.bfloat16), maskbias_t


_ACOUNT_T, _MASKBIAS_T = _build_sample_constants()


def _phase_a(q_ref, k_ref, at_ref, mb_ref, m_ref):
    j = pl.program_id(1)
    q = q_ref[0]
    k = k_ref[0]
    kb = k.astype(jnp.bfloat16)
    qb = q.astype(jnp.bfloat16)
    at = at_ref[...]
    sT = jax.lax.dot_general(kb, qb, (((1,), (1,)), ((), ())),
                             preferred_element_type=jnp.float32)
    mx = jnp.max(sT + mb_ref[...], axis=0)
    ksumT = jax.lax.dot_general(kb, at, (((0,), (0,)), ((), ())),
                                preferred_element_type=jnp.float32)
    ssum = jnp.sum(q.T * ksumT, axis=0)
    m_ref[0, pl.ds(j, 1), :] = (mx - ssum * (1.0 / _L))[None, :]


def _bmax_f32(v, rbuf):
    for sh in (8, 4, 2, 1):
        rbuf[pl.ds(0, 16)] = v
        rbuf[pl.ds(16, 16)] = v
        v = jnp.maximum(v, rbuf[pl.ds(sh, 16)])
    return v


def _bmin_i32(v, ibuf):
    for sh in (8, 4, 2, 1):
        ibuf[pl.ds(0, 16)] = v
        ibuf[pl.ds(16, 16)] = v
        v = jnp.minimum(v, ibuf[pl.ds(sh, 16)])
    return v


def _sc_topk(m_hbm, idx_hbm, mrow, idxrow, rbuf, ibuf):
    wid = lax.axis_index("s") * 2 + lax.axis_index("c")
    lanes = lax.iota(jnp.int32, 16)
    negv = jnp.full((16,), _NEG, jnp.float32)

    @pl.when(wid < _BH)
    def _():
        for i in range(_NT):
            pltpu.sync_copy(m_hbm.at[i, wid], mrow.at[pl.ds(i * _TR, _TR)])

        def t_body(t, carry):
            gmp, lip = carry

            def c_body(c, cc):
                bestv, besti = cc
                v = mrow[pl.ds(c * 16, 16)]
                idxv = lanes + c * 16
                elig = (v < gmp) | ((v == gmp) & (idxv > lip))
                veff = jnp.where(elig, v, negv)
                better = veff > bestv
                return (jnp.where(better, veff, bestv),
                        jnp.where(better, idxv, besti))

            bestv, besti = lax.fori_loop(
                0, _L // 16, c_body, (negv, jnp.zeros((16,), jnp.int32)))
            gm = _bmax_f32(bestv, rbuf)
            cand = jnp.where(bestv == gm, besti,
                             jnp.full((16,), 4096, jnp.int32))
            li = _bmin_i32(cand, ibuf)
            base = (t // 16) * 16
            off = t % 16
            old = idxrow[pl.ds(base, 16)]
            idxrow[pl.ds(base, 16)] = jnp.where(lanes == off, li, old)
            return gm, li

        lax.fori_loop(0, _U, t_body,
                      (jnp.full((16,), 3.0e38, jnp.float32),
                       jnp.full((16,), -1, jnp.int32)))

        pltpu.sync_copy(idxrow, idx_hbm.at[wid])


def _sc_topk_call(m_blk):
    mesh = plsc.VectorSubcoreMesh(core_axis_name="c", subcore_axis_name="s")
    f = functools.partial(
        pl.kernel, mesh=mesh,
        out_type=jax.ShapeDtypeStruct((_BH, 128), jnp.int32),
        scratch_types=[
            pltpu.VMEM((_L,), jnp.float32),
            pltpu.VMEM((128,), jnp.int32),
            pltpu.VMEM((32,), jnp.float32),
            pltpu.VMEM((32,), jnp.int32),
        ],
    )(_sc_topk)
    return f(m_blk)


def _phase_c(idx_ref, q_ref, k_ref, v_ref, o_ref, qr_ref, up_ref):
    j = pl.program_id(0)
    for s2 in range(_U):
        r = idx_ref[j, s2]
        qr_ref[pl.ds(s2, 1), :] = q_ref[0, pl.ds(r, 1), :]
    k = k_ref[0]
    v = v_ref[0]
    sc = jax.lax.dot_general(qr_ref[...].astype(jnp.bfloat16),
                             k.astype(jnp.bfloat16), (((1,), (1,)), ((), ())),
                             preferred_element_type=jnp.float32) * 0.125
    sc = sc - jnp.max(sc, axis=1, keepdims=True)
    e = jnp.exp(sc)
    att = e / jnp.sum(e, axis=1, keepdims=True)
    up_ref[...] = jax.lax.dot_general(att.astype(jnp.bfloat16),
                                      v.astype(jnp.bfloat16),
                                      (((1,), (0,)), ((), ())),
                                      preferred_element_type=jnp.float32)
    vm = jnp.sum(v, axis=0, keepdims=True) * (1.0 / _L)
    o_ref[0] = jnp.broadcast_to(vm, (_L, _D))

    for t in range(_U):
        r = idx_ref[j, t]
        o_ref[0, pl.ds(r, 1), :] = up_ref[pl.ds(t, 1), :]


def kernel(queries, keys, values):
    q3 = queries.reshape(_BH, _L, _D)
    k3 = keys.reshape(_BH, _L, _D)
    v3 = values.reshape(_BH, _L, _D)
    at = jnp.asarray(_ACOUNT_T)
    mb = jnp.asarray(_MASKBIAS_T)

    m_blk = pl.pallas_call(
        _phase_a,
        grid=(_NT, _BH),
        in_specs=[
            pl.BlockSpec((1, _TR, _D), lambda i, j: (j, i, 0)),
            pl.BlockSpec((1, _L, _D), lambda i, j: (j, 0, 0)),
            pl.BlockSpec((_L, _TR), lambda i, j: (0, i)),
            pl.BlockSpec((_L, _TR), lambda i, j: (0, i)),
        ],
        out_specs=pl.BlockSpec((1, _BH, _TR), lambda i, j: (i, 0, 0)),
        out_shape=jax.ShapeDtypeStruct((_NT, _BH, _TR), jnp.float32),
    )(q3, k3, at, mb)

    idx = _sc_topk_call(m_blk)

    ctx = pl.pallas_call(
        _phase_c,
        grid=(_BH,),
        in_specs=[
            pl.BlockSpec(memory_space=pltpu.SMEM),
            pl.BlockSpec((1, _L, _D), lambda j: (j, 0, 0)),
            pl.BlockSpec((1, _L, _D), lambda j: (j, 0, 0)),
            pl.BlockSpec((1, _L, _D), lambda j: (j, 0, 0)),
        ],
        out_specs=pl.BlockSpec((1, _L, _D), lambda j: (j, 0, 0)),
        out_shape=jax.ShapeDtypeStruct((_BH, _L, _D), jnp.float32),
        scratch_shapes=[
            pltpu.VMEM((_U, _D), jnp.float32),
            pltpu.VMEM((_U, _D), jnp.float32),
        ],
    )(idx, q3, k3, v3)

    return ctx.reshape(_B, _H, _L, _D)
